# Initial kernel scaffold; baseline (speedup 1.0000x reference)
import jax
import jax.numpy as jnp
from jax.experimental import pallas as pl


def kernel(hm, dep, wh, rot, dim3d, calib, inv_Ms):
    raise NotImplementedError("write your pallas kernel here")


# speedup vs baseline: 1.4190x; 1.4190x over previous
import functools

import jax
import jax.numpy as jnp
from jax import lax
from jax.experimental import pallas as pl

_PI = 3.141592653589793
_FOCAL_DEFAULT = 1000.0
_TOPK = 100
_NEG = -1e30


def _fmt_angle(a):
    a = jnp.where(a > _PI, a - 2.0 * _PI, a)
    a = jnp.where(a < -_PI, a + 2.0 * _PI, a)
    return a


def _nms_body(hm_ref, score_ref, class_ref):
    c = pl.program_id(1)
    heat = jax.nn.sigmoid(hm_ref[0, 0])
    H, W = heat.shape
    ninf = jnp.full((1, W), _NEG, heat.dtype)
    up = jnp.concatenate([heat[1:, :], ninf], axis=0)
    dn = jnp.concatenate([ninf, heat[:-1, :]], axis=0)
    rowm = jnp.maximum(jnp.maximum(up, dn), heat)
    ninfc = jnp.full((H, 1), _NEG, heat.dtype)
    lf = jnp.concatenate([rowm[:, 1:], ninfc], axis=1)
    rt = jnp.concatenate([ninfc, rowm[:, :-1]], axis=1)
    maxp = jnp.maximum(jnp.maximum(lf, rt), rowm)
    masked = jnp.where(maxp == heat, heat, 0.0)

    @pl.when(c == 0)
    def _init():
        score_ref[0] = masked
        class_ref[0] = jnp.zeros_like(masked, jnp.int32)

    @pl.when(c > 0)
    def _acc():
        prev = score_ref[0]
        better = masked > prev
        score_ref[0] = jnp.where(better, masked, prev)
        class_ref[0] = jnp.where(better, c, class_ref[0])


def _nms_channel_max(hm):
    B, C, H, W = hm.shape
    return pl.pallas_call(
        _nms_body,
        grid=(B, C),
        in_specs=[pl.BlockSpec((1, 1, H, W), lambda b, c: (b, c, 0, 0))],
        out_specs=[
            pl.BlockSpec((1, H, W), lambda b, c: (b, 0, 0)),
            pl.BlockSpec((1, H, W), lambda b, c: (b, 0, 0)),
        ],
        out_shape=[
            jax.ShapeDtypeStruct((B, H, W), jnp.float32),
            jax.ShapeDtypeStruct((B, H, W), jnp.int32),
        ],
    )(hm)


def _decode_body(idx_ref, dep_ref, wh_ref, rot_ref, dim_ref, cal_ref, m_ref,
                 bbox_ref, ctr_ref, dep_o_ref, roty_ref, ax_ref, az_ref,
                 th_ref, loc_ref, W_const):
    idx = idx_ref[...]
    u = (idx % W_const).astype(jnp.float32)
    v = (idx // W_const).astype(jnp.float32)

    def bf(a):
        return a.astype(jnp.bfloat16).astype(jnp.float32)

    m00 = bf(m_ref[0])
    m01 = bf(m_ref[1])
    m02 = bf(m_ref[2])
    m10 = bf(m_ref[3])
    m11 = bf(m_ref[4])
    m12 = bf(m_ref[5])

    def affine(x, y):
        xb = bf(x)
        yb = bf(y)
        return (xb * m00 + yb * m01 + m02, xb * m10 + yb * m11 + m12)

    wh0 = wh_ref[0]
    wh1 = wh_ref[1]
    x1, y1 = affine(u - wh0 * 0.5, v - wh1 * 0.5)
    x2, y2 = affine(u + wh0 * 0.5, v + wh1 * 0.5)
    bbox_ref[0] = x1
    bbox_ref[1] = y1
    bbox_ref[2] = x2
    bbox_ref[3] = y2

    ctx, cty = affine(u, v)
    ctr_ref[0] = ctx
    ctr_ref[1] = cty

    fpx = cal_ref[0]
    cx = cal_ref[1]
    cy = cal_ref[2]
    dep_dec = 1.0 / (jax.nn.sigmoid(dep_ref[...]) + 1e-6) - 1.0
    dep_g = dep_dec * (fpx / _FOCAL_DEFAULT)
    dep_o_ref[...] = dep_g
    loc_x = (ctx - cx) * dep_g / fpx
    loc_y = (cty - cy) * dep_g / fpx
    loc_ref[0] = loc_x
    loc_ref[1] = loc_y
    loc_ref[2] = dep_g

    b0 = rot_ref[0]
    b1 = rot_ref[1]
    b2 = rot_ref[2]
    b3 = rot_ref[3]
    best = b0
    bin_id = jnp.zeros_like(b0, jnp.int32)
    for k, bk in ((1, b1), (2, b2), (3, b3)):
        gt = bk > best
        best = jnp.where(gt, bk, best)
        bin_id = jnp.where(gt, k, bin_id)

    sin_sel = rot_ref[4]
    cos_sel = rot_ref[5]
    for k in (1, 2, 3):
        sel = bin_id == k
        sin_sel = jnp.where(sel, rot_ref[4 + 2 * k], sin_sel)
        cos_sel = jnp.where(sel, rot_ref[5 + 2 * k], cos_sel)
    nrm = jnp.maximum(jnp.sqrt(sin_sel * sin_sel + cos_sel * cos_sel), 1e-12)
    sin_n = sin_sel / nrm
    cos_n = cos_sel / nrm
    centers = jnp.where(bin_id == 0, 0.0,
               jnp.where(bin_id == 1, _PI / 2.0,
                jnp.where(bin_id == 2, _PI, -_PI / 2.0)))
    alpha_z = _fmt_angle(jnp.arctan2(sin_n, cos_n) + centers)
    alpha_x = _fmt_angle(alpha_z - _PI / 2.0)
    theta = _fmt_angle(-jnp.arctan2(loc_x, dep_g))
    roty = _fmt_angle(alpha_x - theta)
    az_ref[...] = alpha_z
    ax_ref[...] = alpha_x
    th_ref[...] = theta
    roty_ref[...] = roty
    _ = dim_ref


def _decode_objects(idx, dep_g, wh_g, rot_g, dim_g, cal3, m6, W):
    B, K = idx.shape
    outs = pl.pallas_call(
        functools.partial(_decode_body, W_const=W),
        out_shape=[
            jax.ShapeDtypeStruct((4, B, K), jnp.float32),
            jax.ShapeDtypeStruct((2, B, K), jnp.float32),
            jax.ShapeDtypeStruct((B, K), jnp.float32),
            jax.ShapeDtypeStruct((B, K), jnp.float32),
            jax.ShapeDtypeStruct((B, K), jnp.float32),
            jax.ShapeDtypeStruct((B, K), jnp.float32),
            jax.ShapeDtypeStruct((B, K), jnp.float32),
            jax.ShapeDtypeStruct((3, B, K), jnp.float32),
        ],
    )(idx, dep_g, wh_g, rot_g, dim_g, cal3, m6)
    return outs


def kernel(hm, dep, wh, rot, dim3d, calib, inv_Ms):
    B, C, H, W = hm.shape
    HW = H * W
    scores_hw, classes_hw = _nms_channel_max(hm)
    scores_all = scores_hw.reshape(B, HW)
    classes_all = classes_hw.reshape(B, HW)

    scores, idx = lax.top_k(scores_all, _TOPK)
    cat_id = jnp.take_along_axis(classes_all, idx, axis=1)

    def gat(x):
        xf = x.reshape(B, x.shape[1], HW)
        g = jnp.take_along_axis(xf, idx[:, None, :], axis=2)
        return jnp.transpose(g, (1, 0, 2))

    dep_g = gat(dep)[0]
    wh_g = gat(wh)
    rot_g = gat(rot)
    dim_g = gat(dim3d)

    norm_focal = (calib[:, 0, 0] + calib[:, 1, 1]) * 0.5
    cal3 = jnp.stack([norm_focal, calib[:, 0, 2], calib[:, 1, 2]])[:, :, None]
    m6 = jnp.stack([inv_Ms[:, 0, 0], inv_Ms[:, 0, 1], inv_Ms[:, 0, 2],
                    inv_Ms[:, 1, 0], inv_Ms[:, 1, 1], inv_Ms[:, 1, 2]])[:, :, None]

    (bboxp, ctrp, dep_o, roty, alpha_x, alpha_z, theta, locp) = _decode_objects(
        idx, dep_g, wh_g, rot_g, dim_g, cal3, m6, W)

    bbox = jnp.transpose(bboxp, (1, 2, 0))
    center_t = jnp.transpose(ctrp, (1, 2, 0))
    loc = jnp.transpose(locp, (1, 2, 0))
    dim_out = jnp.transpose(dim_g, (1, 2, 0))
    return (bbox, scores, cat_id, center_t, dep_o, roty, alpha_x, alpha_z,
            theta, loc, dim_out)


# speedup vs baseline: 2.6855x; 1.8925x over previous
import functools

import jax
import jax.numpy as jnp
from jax import lax
from jax.experimental import pallas as pl
from jax.experimental.pallas import tpu as pltpu
from jax.experimental.pallas import tpu_sc as plsc

_PI = 3.141592653589793
_FOCAL_DEFAULT = 1000.0
_TOPK = 100
_NEG = -1e30
_NEGF = -3.0e38


def _fmt_angle(a):
    a = jnp.where(a > _PI, a - 2.0 * _PI, a)
    a = jnp.where(a < -_PI, a + 2.0 * _PI, a)
    return a


def _nms_body(hm_ref, score_ref, class_ref):
    c = pl.program_id(1)
    heat = jax.nn.sigmoid(hm_ref[0, 0])
    H, W = heat.shape
    ninf = jnp.full((1, W), _NEG, heat.dtype)
    up = jnp.concatenate([heat[1:, :], ninf], axis=0)
    dn = jnp.concatenate([ninf, heat[:-1, :]], axis=0)
    rowm = jnp.maximum(jnp.maximum(up, dn), heat)
    ninfc = jnp.full((H, 1), _NEG, heat.dtype)
    lf = jnp.concatenate([rowm[:, 1:], ninfc], axis=1)
    rt = jnp.concatenate([ninfc, rowm[:, :-1]], axis=1)
    maxp = jnp.maximum(jnp.maximum(lf, rt), rowm)
    masked = jnp.where(maxp == heat, heat, 0.0)

    @pl.when(c == 0)
    def _init():
        score_ref[0] = masked
        class_ref[0] = jnp.zeros_like(masked, jnp.int32)

    @pl.when(c > 0)
    def _acc():
        prev = score_ref[0]
        better = masked > prev
        score_ref[0] = jnp.where(better, masked, prev)
        class_ref[0] = jnp.where(better, c, class_ref[0])


def _nms_channel_max(hm):
    B, C, H, W = hm.shape
    return pl.pallas_call(
        _nms_body,
        grid=(B, C),
        in_specs=[pl.BlockSpec((1, 1, H, W), lambda b, c: (b, c, 0, 0))],
        out_specs=[
            pl.BlockSpec((1, H, W), lambda b, c: (b, 0, 0)),
            pl.BlockSpec((1, H, W), lambda b, c: (b, 0, 0)),
        ],
        out_shape=[
            jax.ShapeDtypeStruct((B, H, W), jnp.float32),
            jax.ShapeDtypeStruct((B, H, W), jnp.int32),
        ],
    )(hm)


_B, _HW = 8, 192 * 480
_SEG = _HW // 4
_NJ = _SEG // 256
_NJP = 96
_KP = 112


_IMAX = 2147483647


def _argmax16(vref, base16, n, idx0, istep):
    acc = vref[pl.ds(base16, 16)]
    ai = jnp.zeros((16,), jnp.int32) + idx0
    for k in range(1, n):
        v = vref[pl.ds(base16 + k * 16, 16)]
        up = v > acc
        acc = jnp.where(up, v, acc)
        ai = jnp.where(up, idx0 + k * istep, ai)
    return acc, ai


def _argmax16i(vref, iref, base16, n):
    acc = vref[pl.ds(base16, 16)]
    ai = iref[pl.ds(base16, 16)]
    for k in range(1, n):
        v = vref[pl.ds(base16 + k * 16, 16)]
        vi = iref[pl.ds(base16 + k * 16, 16)]
        up = v > acc
        acc = jnp.where(up, v, acc)
        ai = jnp.where(up, vi, ai)
    return acc, ai


def _sc_body(scores_hbm, classes_hbm, dep_hbm, wh_hbm, rot_hbm, dim_hbm,
             scores_o, idx_o, cat_o, dep_o, wh_o, rot_o, dim_o,
             data_v, l1_v, li1_v, l2_v, li2_v, val_v, ind_v, shv, shi,
             mval_v, mind_v, win_dep, win_cls, win_wh, win_rot, win_dim,
             cls_rows, dep_rows, wh_rows, rot_rows, dim_rows, sem):
    cid = lax.axis_index("c")
    sid = lax.axis_index("s")
    b = cid * 4 + sid // 4
    part = sid % 4
    base = part * _SEG
    iota = jnp.arange(16, dtype=jnp.int32)
    negf = jnp.full((16,), _NEGF, jnp.float32)

    pltpu.sync_copy(scores_hbm.at[b, pl.ds(base, _SEG)], data_v)

    def _build_l1(j, carry):
        acc, ai = _argmax16(data_v, j * 256, 16, j * 16, 1)
        l1_v[pl.ds(j * 16, 16)] = acc
        li1_v[pl.ds(j * 16, 16)] = ai
        return carry
    lax.fori_loop(0, _NJ, _build_l1, 0)
    for j in range(_NJ, _NJP):
        l1_v[pl.ds(j * 16, 16)] = negf
        li1_v[pl.ds(j * 16, 16)] = jnp.zeros((16,), jnp.int32)
    for g in range(6):
        acc, ai = _argmax16i(l1_v, li1_v, g * 256, 16)
        l2_v[pl.ds(g * 16, 16)] = acc
        li2_v[pl.ds(g * 16, 16)] = ai
    for q in range(_KP // 16):
        val_v[pl.ds(q * 16, 16)] = negf
        ind_v[pl.ds(q * 16, 16)] = jnp.zeros((16,), jnp.int32)

    def _extract(i, carry):
        l3, vmin = _argmax16i(l2_v, li2_v, 0, 6)
        m = jnp.max(l3)
        e_cand = jnp.where(l3 == m, vmin * 16 + iota, _IMAX)
        e = jnp.min(e_cand)
        lane = e % 16
        dvreg = e // 16
        j2 = dvreg // 16
        g2 = j2 // 16
        lane_sel = iota == lane
        vk = data_v[pl.ds(dvreg * 16, 16)]
        data_v[pl.ds(dvreg * 16, 16)] = jnp.where(lane_sel, negf, vk)
        acc, ai = _argmax16(data_v, j2 * 256, 16, j2 * 16, 1)
        l1_v[pl.ds(j2 * 16, 16)] = acc
        li1_v[pl.ds(j2 * 16, 16)] = ai
        acc, ai = _argmax16i(l1_v, li1_v, g2 * 256, 16)
        l2_v[pl.ds(g2 * 16, 16)] = acc
        li2_v[pl.ds(g2 * 16, 16)] = ai
        q16 = (i // 16) * 16
        wsel = iota == (i % 16)
        val_v[pl.ds(q16, 16)] = jnp.where(wsel, m, val_v[pl.ds(q16, 16)])
        ind_v[pl.ds(q16, 16)] = jnp.where(wsel, base + e,
                                          ind_v[pl.ds(q16, 16)])
        return carry
    lax.fori_loop(0, _TOPK, _extract, 0)

    pltpu.sync_copy(val_v, shv.at[sid])
    pltpu.sync_copy(ind_v, shi.at[sid])
    plsc.subcore_barrier()

    @pl.when(part == 0)
    def _merge_and_gather():
        for r in range(4):
            pltpu.sync_copy(shv.at[sid + r], mval_v.at[pl.ds(r * _KP, _KP)])
            pltpu.sync_copy(shi.at[sid + r], mind_v.at[pl.ds(r * _KP, _KP)])

        nv = 4 * _KP // 16

        def _extract2(i, carry):
            m3 = mval_v[pl.ds(0, 16)]
            for k in range(1, nv):
                m3 = jnp.maximum(m3, mval_v[pl.ds(k * 16, 16)])
            m = jnp.max(m3)
            gm = jnp.full((16,), _IMAX, jnp.int32)
            for k in range(nv):
                vv = mval_v[pl.ds(k * 16, 16)]
                mi = mind_v[pl.ds(k * 16, 16)]
                gm = jnp.minimum(gm, jnp.where(vv == m, mi, _IMAX))
            gidx = jnp.min(gm)
            for k in range(nv):
                vv = mval_v[pl.ds(k * 16, 16)]
                mi = mind_v[pl.ds(k * 16, 16)]
                hit = (vv == m) & (mi == gidx)
                mval_v[pl.ds(k * 16, 16)] = jnp.where(hit, negf, vv)
            q16 = (i // 16) * 16
            wsel = iota == (i % 16)
            val_v[pl.ds(q16, 16)] = jnp.where(wsel, m, val_v[pl.ds(q16, 16)])
            ind_v[pl.ds(q16, 16)] = jnp.where(wsel, gidx,
                                              ind_v[pl.ds(q16, 16)])
            a8 = pl.multiple_of((gidx // 8) * 8, 8)
            d8 = pl.ds(pl.multiple_of(i * 8, 8), 8)
            pltpu.async_copy(dep_hbm.at[b, pl.ds(a8, 8)],
                             win_dep.at[d8], sem)
            pltpu.async_copy(classes_hbm.at[b, pl.ds(a8, 8)],
                             win_cls.at[d8], sem)
            for c in range(2):
                pltpu.async_copy(wh_hbm.at[b * 2 + c, pl.ds(a8, 8)],
                                 win_wh.at[pl.ds(pl.multiple_of(c * _KP * 8 + i * 8, 8), 8)], sem)
            for c in range(12):
                pltpu.async_copy(rot_hbm.at[b * 12 + c, pl.ds(a8, 8)],
                                 win_rot.at[pl.ds(pl.multiple_of(c * _KP * 8 + i * 8, 8), 8)], sem)
            for c in range(3):
                pltpu.async_copy(dim_hbm.at[b * 3 + c, pl.ds(a8, 8)],
                                 win_dim.at[pl.ds(pl.multiple_of(c * _KP * 8 + i * 8, 8), 8)], sem)
            return carry
        lax.fori_loop(0, _TOPK, _extract2, 0)

        n8 = _TOPK * 8
        pltpu.make_async_copy(dep_hbm.at[0, pl.ds(0, n8)],
                              win_dep.at[pl.ds(0, n8)], sem).wait()
        pltpu.make_async_copy(classes_hbm.at[0, pl.ds(0, n8)],
                              win_cls.at[pl.ds(0, n8)], sem).wait()
        for c in range(2):
            pltpu.make_async_copy(wh_hbm.at[0, pl.ds(0, n8)],
                                  win_wh.at[pl.ds(c * _KP * 8, n8)], sem).wait()
        for c in range(12):
            pltpu.make_async_copy(rot_hbm.at[0, pl.ds(0, n8)],
                                  win_rot.at[pl.ds(c * _KP * 8, n8)], sem).wait()
        for c in range(3):
            pltpu.make_async_copy(dim_hbm.at[0, pl.ds(0, n8)],
                                  win_dim.at[pl.ds(c * _KP * 8, n8)], sem).wait()

        for q in range(_KP // 16):
            j16 = iota + q * 16
            off16 = j16 * 8 + ind_v[pl.ds(q * 16, 16)] % 8
            dep_rows[pl.ds(q * 16, 16)] = plsc.load_gather(win_dep, [off16])
            cls_rows[pl.ds(q * 16, 16)] = plsc.load_gather(win_cls, [off16])
            for c in range(2):
                wh_rows[c, pl.ds(q * 16, 16)] = plsc.load_gather(
                    win_wh, [off16 + c * _KP * 8])
            for c in range(12):
                rot_rows[c, pl.ds(q * 16, 16)] = plsc.load_gather(
                    win_rot, [off16 + c * _KP * 8])
            for c in range(3):
                dim_rows[c, pl.ds(q * 16, 16)] = plsc.load_gather(
                    win_dim, [off16 + c * _KP * 8])

        pltpu.sync_copy(val_v, scores_o.at[b])
        pltpu.sync_copy(ind_v, idx_o.at[b])
        pltpu.sync_copy(cls_rows, cat_o.at[b])
        pltpu.sync_copy(dep_rows, dep_o.at[b])
        pltpu.sync_copy(wh_rows, wh_o.at[b])
        pltpu.sync_copy(rot_rows, rot_o.at[b])
        pltpu.sync_copy(dim_rows, dim_o.at[b])


def _sc_topk_gather(scores2d, classes_t, dep_t, wh_t, rot_t, dim_t):
    f32, i32 = jnp.float32, jnp.int32
    mesh = plsc.VectorSubcoreMesh(core_axis_name="c", subcore_axis_name="s")
    return pl.kernel(
        _sc_body,
        out_type=[
            jax.ShapeDtypeStruct((_B, _KP), f32),
            jax.ShapeDtypeStruct((_B, _KP), i32),
            jax.ShapeDtypeStruct((_B, _KP), i32),
            jax.ShapeDtypeStruct((_B, _KP), f32),
            jax.ShapeDtypeStruct((_B, 2, _KP), f32),
            jax.ShapeDtypeStruct((_B, 12, _KP), f32),
            jax.ShapeDtypeStruct((_B, 3, _KP), f32),
        ],
        mesh=mesh,
        compiler_params=pltpu.CompilerParams(needs_layout_passes=False,
                                             use_tc_tiling_on_sc=False),
        scratch_types=[
            pltpu.VMEM((_SEG,), f32),
            pltpu.VMEM((_NJP * 16,), f32),
            pltpu.VMEM((_NJP * 16,), i32),
            pltpu.VMEM((6 * 16,), f32),
            pltpu.VMEM((6 * 16,), i32),
            pltpu.VMEM((_KP,), f32),
            pltpu.VMEM((_KP,), i32),
            pltpu.VMEM_SHARED((16, _KP), f32),
            pltpu.VMEM_SHARED((16, _KP), i32),
            pltpu.VMEM((4 * _KP,), f32),
            pltpu.VMEM((4 * _KP,), i32),
            pltpu.VMEM((_KP * 8,), f32),
            pltpu.VMEM((_KP * 8,), i32),
            pltpu.VMEM((2 * _KP * 8,), f32),
            pltpu.VMEM((12 * _KP * 8,), f32),
            pltpu.VMEM((3 * _KP * 8,), f32),
            pltpu.VMEM((_KP,), i32),
            pltpu.VMEM((_KP,), f32),
            pltpu.VMEM((2, _KP), f32),
            pltpu.VMEM((12, _KP), f32),
            pltpu.VMEM((3, _KP), f32),
            pltpu.SemaphoreType.DMA,
        ],
    )(scores2d, classes_t, dep_t, wh_t, rot_t, dim_t)


def _decode_body(idx_ref, dep_ref, wh_ref, rot_ref, dim_ref, cal_ref, m_ref,
                 bbox_ref, ctr_ref, dep_o_ref, roty_ref, ax_ref, az_ref,
                 th_ref, loc_ref, W_const):
    idx = idx_ref[...]
    u = (idx % W_const).astype(jnp.float32)
    v = (idx // W_const).astype(jnp.float32)

    def bf(a):
        return a.astype(jnp.bfloat16).astype(jnp.float32)

    m00 = bf(m_ref[0])
    m01 = bf(m_ref[1])
    m02 = bf(m_ref[2])
    m10 = bf(m_ref[3])
    m11 = bf(m_ref[4])
    m12 = bf(m_ref[5])

    def affine(x, y):
        xb = bf(x)
        yb = bf(y)
        return (xb * m00 + yb * m01 + m02, xb * m10 + yb * m11 + m12)

    wh0 = wh_ref[0]
    wh1 = wh_ref[1]
    x1, y1 = affine(u - wh0 * 0.5, v - wh1 * 0.5)
    x2, y2 = affine(u + wh0 * 0.5, v + wh1 * 0.5)
    bbox_ref[0] = x1
    bbox_ref[1] = y1
    bbox_ref[2] = x2
    bbox_ref[3] = y2

    ctx, cty = affine(u, v)
    ctr_ref[0] = ctx
    ctr_ref[1] = cty

    fpx = cal_ref[0]
    cx = cal_ref[1]
    cy = cal_ref[2]
    dep_dec = 1.0 / (jax.nn.sigmoid(dep_ref[...]) + 1e-6) - 1.0
    dep_g = dep_dec * (fpx / _FOCAL_DEFAULT)
    dep_o_ref[...] = dep_g
    loc_x = (ctx - cx) * dep_g / fpx
    loc_y = (cty - cy) * dep_g / fpx
    loc_ref[0] = loc_x
    loc_ref[1] = loc_y
    loc_ref[2] = dep_g

    b0 = rot_ref[0]
    b1 = rot_ref[1]
    b2 = rot_ref[2]
    b3 = rot_ref[3]
    best = b0
    bin_id = jnp.zeros_like(b0, jnp.int32)
    for k, bk in ((1, b1), (2, b2), (3, b3)):
        gt = bk > best
        best = jnp.where(gt, bk, best)
        bin_id = jnp.where(gt, k, bin_id)

    sin_sel = rot_ref[4]
    cos_sel = rot_ref[5]
    for k in (1, 2, 3):
        sel = bin_id == k
        sin_sel = jnp.where(sel, rot_ref[4 + 2 * k], sin_sel)
        cos_sel = jnp.where(sel, rot_ref[5 + 2 * k], cos_sel)
    nrm = jnp.maximum(jnp.sqrt(sin_sel * sin_sel + cos_sel * cos_sel), 1e-12)
    sin_n = sin_sel / nrm
    cos_n = cos_sel / nrm
    centers = jnp.where(bin_id == 0, 0.0,
               jnp.where(bin_id == 1, _PI / 2.0,
                jnp.where(bin_id == 2, _PI, -_PI / 2.0)))
    alpha_z = _fmt_angle(jnp.arctan2(sin_n, cos_n) + centers)
    alpha_x = _fmt_angle(alpha_z - _PI / 2.0)
    theta = _fmt_angle(-jnp.arctan2(loc_x, dep_g))
    roty = _fmt_angle(alpha_x - theta)
    az_ref[...] = alpha_z
    ax_ref[...] = alpha_x
    th_ref[...] = theta
    roty_ref[...] = roty
    _ = dim_ref


def _decode_objects(idx, dep_g, wh_g, rot_g, dim_g, cal3, m6, W):
    B, K = idx.shape
    outs = pl.pallas_call(
        functools.partial(_decode_body, W_const=W),
        out_shape=[
            jax.ShapeDtypeStruct((4, B, K), jnp.float32),
            jax.ShapeDtypeStruct((2, B, K), jnp.float32),
            jax.ShapeDtypeStruct((B, K), jnp.float32),
            jax.ShapeDtypeStruct((B, K), jnp.float32),
            jax.ShapeDtypeStruct((B, K), jnp.float32),
            jax.ShapeDtypeStruct((B, K), jnp.float32),
            jax.ShapeDtypeStruct((B, K), jnp.float32),
            jax.ShapeDtypeStruct((3, B, K), jnp.float32),
        ],
    )(idx, dep_g, wh_g, rot_g, dim_g, cal3, m6)
    return outs


def kernel(hm, dep, wh, rot, dim3d, calib, inv_Ms):
    B, C, H, W = hm.shape
    HW = H * W
    scores_hw, classes_hw = _nms_channel_max(hm)
    scores_all = scores_hw.reshape(B, HW)

    (scores_p, idx_p, cat_p, dep_p, wh_p, rot_p, dim_p) = _sc_topk_gather(
        scores_all,
        classes_hw.reshape(B, HW),
        dep.reshape(B, HW),
        wh.reshape(B * 2, HW),
        rot.reshape(B * 12, HW),
        dim3d.reshape(B * 3, HW),
    )
    scores = scores_p[:, :_TOPK]
    idx = idx_p[:, :_TOPK]
    cat_id = cat_p[:, :_TOPK]
    dep_g = dep_p[:, :_TOPK]
    wh_g = jnp.transpose(wh_p[:, :, :_TOPK], (1, 0, 2))
    rot_g = jnp.transpose(rot_p[:, :, :_TOPK], (1, 0, 2))
    dim_g = jnp.transpose(dim_p[:, :, :_TOPK], (1, 0, 2))

    norm_focal = (calib[:, 0, 0] + calib[:, 1, 1]) * 0.5
    cal3 = jnp.stack([norm_focal, calib[:, 0, 2], calib[:, 1, 2]])[:, :, None]
    m6 = jnp.stack([inv_Ms[:, 0, 0], inv_Ms[:, 0, 1], inv_Ms[:, 0, 2],
                    inv_Ms[:, 1, 0], inv_Ms[:, 1, 1], inv_Ms[:, 1, 2]])[:, :, None]

    (bboxp, ctrp, dep_o, roty, alpha_x, alpha_z, theta, locp) = _decode_objects(
        idx, dep_g, wh_g, rot_g, dim_g, cal3, m6, W)

    bbox = jnp.transpose(bboxp, (1, 2, 0))
    center_t = jnp.transpose(ctrp, (1, 2, 0))
    loc = jnp.transpose(locp, (1, 2, 0))
    dim_out = jnp.transpose(dim_g, (1, 2, 0))
    return (bbox, scores, cat_id, center_t, dep_o, roty, alpha_x, alpha_z,
            theta, loc, dim_out)
